# baseline (speedup 1.0000x reference)
import functools

import jax
import jax.numpy as jnp
from jax import lax
from jax.experimental import pallas as pl
from jax.experimental.pallas import tpu as pltpu
from jax.experimental.pallas import tpu_sc as plsc

_PTS = 64
_IMP = 64
_EPS = 1e-5
_CH = 64
_L = 16


def _sc_call(n, a_f, b1_f, b2_f):
    info = plsc.get_sparse_core_info()
    nc, ns = info.num_cores, info.num_subcores
    nw = nc * ns
    rpw = n // nw
    nch = rpw // _CH
    npair = nch // 2
    groups = _CH // _L
    csz = _CH * 128
    cdfg = _PTS * _L
    histg = (_IMP + 2) * _L

    mesh = plsc.VectorSubcoreMesh(core_axis_name="c", subcore_axis_name="s")

    @functools.partial(
        pl.kernel,
        out_type=[jax.ShapeDtypeStruct((n * 128,), jnp.float32),
                  jax.ShapeDtypeStruct((n * 128,), jnp.float32)],
        mesh=mesh,
        scratch_types=[
            pltpu.VMEM((2 * csz,), jnp.float32),
            pltpu.VMEM((2 * csz,), jnp.float32),
            pltpu.VMEM((2 * csz,), jnp.float32),
            pltpu.VMEM((2 * csz,), jnp.float32),
            pltpu.VMEM((2 * csz,), jnp.float32),
            pltpu.VMEM((4 * cdfg,), jnp.float32),
            pltpu.VMEM((4 * histg,), jnp.int32),
            pltpu.SemaphoreType.DMA,
            pltpu.SemaphoreType.DMA,
            pltpu.SemaphoreType.DMA,
            pltpu.SemaphoreType.DMA,
            pltpu.SemaphoreType.DMA,
            pltpu.SemaphoreType.DMA,
            pltpu.SemaphoreType.DMA,
            pltpu.SemaphoreType.DMA,
        ],
        compiler_params=pltpu.CompilerParams(needs_layout_passes=False),
    )
    def body(a_h, b1_h, b2_h, y1_h, y2_h,
             a_vm, b1_vm, b2_vm, y1_vm, y2_vm, cdf_vm, hist_vm,
             sa0, sa1, sb0, sb1, sc0, sc1, so0, so1):
        wid = lax.axis_index("s") * nc + lax.axis_index("c")
        iota = lax.iota(jnp.int32, _L)
        zf = jnp.zeros((_L,), jnp.float32)
        onef = jnp.ones((_L,), jnp.float32)
        onei = jnp.ones((_L,), jnp.int32)
        zi = jnp.zeros((_L,), jnp.int32)
        sa = (sa0, sa1)
        sb = (sb0, sb1)
        sc = (sc0, sc1)
        so = (so0, so1)

        def in_copies(ch, slot):
            off = wid * rpw * 128 + ch * csz
            hb = pl.ds(off, csz)
            vm = pl.ds(slot * csz, csz)
            return (
                pltpu.make_async_copy(a_h.at[hb], a_vm.at[vm], sa[slot]),
                pltpu.make_async_copy(b1_h.at[hb], b1_vm.at[vm], sb[slot]),
                pltpu.make_async_copy(b2_h.at[hb], b2_vm.at[vm], sc[slot]),
            )

        def out_copies(ch, slot):
            off = wid * rpw * 128 + ch * csz
            hb = pl.ds(off, csz)
            vm = pl.ds(slot * csz, csz)
            return (
                pltpu.make_async_copy(y1_vm.at[vm], y1_h.at[hb], so[slot]),
                pltpu.make_async_copy(y2_vm.at[vm], y2_h.at[hb], so[slot]),
            )

        def compute_group(g, slot):
            voff = slot * csz
            ray = g * _L + iota
            ray_p = voff + ray * 128
            coff = g * cdfg
            hoff = g * histg
            @plsc.parallel_loop(0, _IMP + 2, unroll=8)
            def hzero(v):
                hist_vm[pl.ds(hoff + v * _L, _L)] = zi

            ob = ray_p + 64
            o0 = plsc.load_gather(b2_vm, [ob])
            o1 = plsc.load_gather(b2_vm, [ob + 1])
            o2 = plsc.load_gather(b2_vm, [ob + 2])
            e0 = plsc.load_gather(b2_vm, [ob + 3])
            e1 = plsc.load_gather(b2_vm, [ob + 4])
            e2 = plsc.load_gather(b2_vm, [ob + 5])

            @plsc.parallel_loop(0, _PTS, unroll=8,
                               carry=(onef, zf, zf, zf, zf))
            def march(k, carry):
                trans, cdf, a0, a1, a2 = carry
                opk = plsc.load_gather(a_vm, [ray_p + k])
                w = opk * trans
                trans = trans * (1.0 - opk)
                cdf = cdf + (w + _EPS)
                cdf_vm[pl.ds(coff + k * _L, _L)] = cdf
                vb = ray_p + k
                a0 = a0 + w * plsc.load_gather(b1_vm, [vb])
                a1 = a1 + w * plsc.load_gather(b1_vm, [vb + 64])
                a2 = a2 + w * plsc.load_gather(b2_vm, [vb])
                return trans, cdf, a0, a1, a2

            _, ctot, a0, a1, a2 = march
            acc_o = jnp.clip(ctot - _PTS * _EPS, 0.0, 1.0)
            ya = ray_p + 64
            plsc.store_scatter(y2_vm, [ya], a0)
            plsc.store_scatter(y2_vm, [ya + 1], a1)
            plsc.store_scatter(y2_vm, [ya + 2], a2)
            plsc.store_scatter(y2_vm, [ya + 3], acc_o)

            scale = jnp.float32(_IMP) / ctot

            @plsc.parallel_loop(0, _PTS, unroll=8)
            def bink(k):
                ck = cdf_vm[pl.ds(coff + k * _L, _L)]
                x = ck * scale
                xi = x.astype(jnp.int32)
                xi = xi + (xi.astype(jnp.float32) < x).astype(jnp.int32)
                m = jnp.minimum(xi, _IMP + 1)
                plsc.addupdate_scatter(hist_vm, [hoff + m * _L + iota], onei)

            dep = ray_p + 64
            c_first = cdf_vm[pl.ds(coff, _L)]
            d_first = plsc.load_gather(a_vm, [dep])
            d_last = plsc.load_gather(a_vm, [dep + (_PTS - 1)])

            @plsc.parallel_loop(1, _IMP + 1, unroll=8,
                               carry=(d_first, hist_vm[pl.ds(hoff, _L)]))
            def sample(j, carry):
                f_prev, cnt = carry
                cnt = cnt + hist_vm[pl.ds(hoff + j * _L, _L)]
                i = jnp.clip(cnt, 1, _PTS - 1)
                g0 = coff + (i - 1) * _L + iota
                c0 = plsc.load_gather(cdf_vm, [g0])
                c1 = plsc.load_gather(cdf_vm, [g0 + _L])
                di = dep + (i - 1)
                d0 = plsc.load_gather(a_vm, [di])
                d1 = plsc.load_gather(a_vm, [di + 1])
                uj = lax.convert_element_type(j, jnp.float32) * (1.0 / _IMP)
                u = uj * ctot
                f = d0 + ((u - c0) / (c1 - c0)) * (d1 - d0)
                f = jnp.where(u < c_first, d_first, f)
                f = jnp.where(u >= ctot, d_last, f)
                mid = 0.5 * (f_prev + f)
                yb = ray_p + (j - 1)
                plsc.store_scatter(y1_vm, [yb], o0 + mid * e0)
                plsc.store_scatter(y1_vm, [yb + 64], o1 + mid * e1)
                plsc.store_scatter(y2_vm, [yb], o2 + mid * e2)
                return f, cnt

            del sample

        def compute_chunk(slot):
            @plsc.parallel_loop(0, groups, unroll=4)
            def grp(g):
                compute_group(g, slot)

        def half(p, ch, slot):
            ca, cb, cc = in_copies(ch, slot)
            ca.wait()
            cb.wait()
            cc.wait()
            @pl.when(p > 0)
            def _():
                oa, ob_ = out_copies(ch - 2, slot)
                oa.wait()
                ob_.wait()

            compute_chunk(slot)
            oa, ob_ = out_copies(ch, slot)
            oa.start()
            ob_.start()
            @pl.when(ch + 2 < nch)
            def _():
                na, nb, ncp = in_copies(ch + 2, slot)
                na.start()
                nb.start()
                ncp.start()

        def pair_body(p, _):
            ch0 = p * 2
            half(p, ch0, 0)
            half(p, ch0 + 1, 1)
            return 0

        pa, pb, pc = in_copies(0, 0)
        pa.start()
        pb.start()
        pc.start()
        qa, qb, qc = in_copies(1, 1)
        qa.start()
        qb.start()
        qc.start()
        lax.fori_loop(0, npair, pair_body, 0)
        fa, fb = out_copies(nch - 2, 0)
        fa.wait()
        fb.wait()
        ga, gb = out_copies(nch - 1, 1)
        ga.wait()
        gb.wait()

    return body(a_f, b1_f, b2_f)


def kernel(opacities, values, depths, origins, dirs):
    n = opacities.shape[0]
    a2 = (jnp.pad(opacities, ((0, 0), (0, 64)))
          + jnp.pad(depths, ((0, 0), (64, 0))))
    b1 = (jnp.pad(values[:, :, 0], ((0, 0), (0, 64)))
          + jnp.pad(values[:, :, 1], ((0, 0), (64, 0))))
    b2 = (jnp.pad(values[:, :, 2], ((0, 0), (0, 64)))
          + jnp.pad(origins, ((0, 0), (64, 61)))
          + jnp.pad(dirs, ((0, 0), (67, 58))))
    y1, y2 = _sc_call(n, a2.reshape(-1), b1.reshape(-1), b2.reshape(-1))
    y1 = y1.reshape(n, 128)
    y2 = y2.reshape(n, 128)
    accv = y2[:, 64:67][:, None, :]
    acco = jnp.broadcast_to(y2[:, 67:68][:, None, :], (n, 1, 3))
    coords = jnp.stack([y1[:, 0:64], y1[:, 64:128], y2[:, 0:64]], axis=-1)
    return jnp.concatenate([accv, acco, coords], axis=1)


# speedup vs baseline: 1.0071x; 1.0071x over previous
import functools

import jax
import jax.numpy as jnp
from jax import lax
from jax.experimental import pallas as pl
from jax.experimental.pallas import tpu as pltpu
from jax.experimental.pallas import tpu_sc as plsc

_PTS = 64
_IMP = 64
_EPS = 1e-5
_CH = 64
_L = 16


def _sc_call(n, op2, dep2, b1_f, b2_f):
    info = plsc.get_sparse_core_info()
    nc, ns = info.num_cores, info.num_subcores
    nw = nc * ns
    rpw = n // nw
    nch = rpw // _CH
    npair = nch // 2
    groups = _CH // _L
    csz = _CH * 128
    cdfg = _PTS * _L
    histg = (_IMP + 2) * _L

    mesh = plsc.VectorSubcoreMesh(core_axis_name="c", subcore_axis_name="s")

    @functools.partial(
        pl.kernel,
        out_type=[jax.ShapeDtypeStruct((n * 128,), jnp.float32),
                  jax.ShapeDtypeStruct((n * 128,), jnp.float32)],
        mesh=mesh,
        scratch_types=[
            pltpu.VMEM((2 * _CH, _PTS), jnp.float32),
            pltpu.VMEM((2 * _CH, _PTS), jnp.float32),
            pltpu.VMEM((2 * csz,), jnp.float32),
            pltpu.VMEM((2 * csz,), jnp.float32),
            pltpu.VMEM((2 * csz,), jnp.float32),
            pltpu.VMEM((2 * csz,), jnp.float32),
            pltpu.VMEM((4 * cdfg,), jnp.float32),
            pltpu.VMEM((4 * histg,), jnp.int32),
            pltpu.SemaphoreType.DMA,
            pltpu.SemaphoreType.DMA,
            pltpu.SemaphoreType.DMA,
            pltpu.SemaphoreType.DMA,
            pltpu.SemaphoreType.DMA,
            pltpu.SemaphoreType.DMA,
            pltpu.SemaphoreType.DMA,
            pltpu.SemaphoreType.DMA,
        ],
        compiler_params=pltpu.CompilerParams(needs_layout_passes=False),
    )
    def body(op_h, dep_h, b1_h, b2_h, y1_h, y2_h,
             op_vm, dep_vm, b1_vm, b2_vm, y1_vm, y2_vm, cdf_vm, hist_vm,
             sa0, sa1, sb0, sb1, sc0, sc1, so0, so1):
        wid = lax.axis_index("s") * nc + lax.axis_index("c")
        iota = lax.iota(jnp.int32, _L)
        zf = jnp.zeros((_L,), jnp.float32)
        onef = jnp.ones((_L,), jnp.float32)
        onei = jnp.ones((_L,), jnp.int32)
        zi = jnp.zeros((_L,), jnp.int32)
        sa = (sa0, sa1)
        sb = (sb0, sb1)
        sc = (sc0, sc1)
        so = (so0, so1)

        def in_copies(ch, slot):
            row = wid * rpw + ch * _CH
            off = wid * rpw * 128 + ch * csz
            hb = pl.ds(off, csz)
            vm = pl.ds(slot * csz, csz)
            rows = pl.ds(row, _CH)
            vrows = pl.ds(slot * _CH, _CH)
            return (
                pltpu.make_async_copy(op_h.at[rows], op_vm.at[vrows],
                                      sa[slot]),
                pltpu.make_async_copy(dep_h.at[rows], dep_vm.at[vrows],
                                      sa[slot]),
                pltpu.make_async_copy(b1_h.at[hb], b1_vm.at[vm], sb[slot]),
                pltpu.make_async_copy(b2_h.at[hb], b2_vm.at[vm], sc[slot]),
            )

        def out_copies(ch, slot):
            off = wid * rpw * 128 + ch * csz
            hb = pl.ds(off, csz)
            vm = pl.ds(slot * csz, csz)
            return (
                pltpu.make_async_copy(y1_vm.at[vm], y1_h.at[hb], so[slot]),
                pltpu.make_async_copy(y2_vm.at[vm], y2_h.at[hb], so[slot]),
            )

        def compute_group(g, slot):
            voff = slot * csz
            ray = g * _L + iota
            ray_p = voff + ray * 128
            row = slot * _CH + ray
            coff = g * cdfg
            hoff = g * histg
            @plsc.parallel_loop(0, _IMP + 2, unroll=8)
            def hzero(v):
                hist_vm[pl.ds(hoff + v * _L, _L)] = zi

            ob = ray_p + 64
            o0 = plsc.load_gather(b2_vm, [ob])
            o1 = plsc.load_gather(b2_vm, [ob + 1])
            o2 = plsc.load_gather(b2_vm, [ob + 2])
            e0 = plsc.load_gather(b2_vm, [ob + 3])
            e1 = plsc.load_gather(b2_vm, [ob + 4])
            e2 = plsc.load_gather(b2_vm, [ob + 5])

            @plsc.parallel_loop(0, _PTS, unroll=8,
                               carry=(onef, zf, zf, zf, zf))
            def march(k, carry):
                trans, cdf, a0, a1, a2 = carry
                opk = plsc.load_gather(op_vm, [row, zi + k])
                w = opk * trans
                trans = trans * (1.0 - opk)
                cdf = cdf + (w + _EPS)
                cdf_vm[pl.ds(coff + k * _L, _L)] = cdf
                vb = ray_p + k
                a0 = a0 + w * plsc.load_gather(b1_vm, [vb])
                a1 = a1 + w * plsc.load_gather(b1_vm, [vb + 64])
                a2 = a2 + w * plsc.load_gather(b2_vm, [vb])
                return trans, cdf, a0, a1, a2

            _, ctot, a0, a1, a2 = march
            acc_o = jnp.clip(ctot - _PTS * _EPS, 0.0, 1.0)
            ya = ray_p + 64
            plsc.store_scatter(y2_vm, [ya], a0)
            plsc.store_scatter(y2_vm, [ya + 1], a1)
            plsc.store_scatter(y2_vm, [ya + 2], a2)
            plsc.store_scatter(y2_vm, [ya + 3], acc_o)

            scale = jnp.float32(_IMP) / ctot

            @plsc.parallel_loop(0, _PTS, unroll=8)
            def bink(k):
                ck = cdf_vm[pl.ds(coff + k * _L, _L)]
                x = ck * scale
                xi = x.astype(jnp.int32)
                xi = xi + (xi.astype(jnp.float32) < x).astype(jnp.int32)
                m = jnp.minimum(xi, _IMP + 1)
                plsc.addupdate_scatter(hist_vm, [hoff + m * _L + iota], onei)

            c_first = cdf_vm[pl.ds(coff, _L)]
            d_first = plsc.load_gather(dep_vm, [row, zi])
            d_last = plsc.load_gather(dep_vm, [row, zi + (_PTS - 1)])

            @plsc.parallel_loop(1, _IMP + 1, unroll=8,
                               carry=(d_first, hist_vm[pl.ds(hoff, _L)]))
            def sample(j, carry):
                f_prev, cnt = carry
                cnt = cnt + hist_vm[pl.ds(hoff + j * _L, _L)]
                i = jnp.clip(cnt, 1, _PTS - 1)
                g0 = coff + (i - 1) * _L + iota
                c0 = plsc.load_gather(cdf_vm, [g0])
                c1 = plsc.load_gather(cdf_vm, [g0 + _L])
                d0 = plsc.load_gather(dep_vm, [row, i - 1])
                d1 = plsc.load_gather(dep_vm, [row, i])
                uj = lax.convert_element_type(j, jnp.float32) * (1.0 / _IMP)
                u = uj * ctot
                f = d0 + ((u - c0) / (c1 - c0)) * (d1 - d0)
                f = jnp.where(u < c_first, d_first, f)
                f = jnp.where(u >= ctot, d_last, f)
                mid = 0.5 * (f_prev + f)
                yb = ray_p + (j - 1)
                plsc.store_scatter(y1_vm, [yb], o0 + mid * e0)
                plsc.store_scatter(y1_vm, [yb + 64], o1 + mid * e1)
                plsc.store_scatter(y2_vm, [yb], o2 + mid * e2)
                return f, cnt

            del sample

        def compute_chunk(slot):
            @plsc.parallel_loop(0, groups, unroll=2)
            def grp(g):
                compute_group(g, slot)

        def half(p, ch, slot):
            ca, cd, cb, cc = in_copies(ch, slot)
            ca.wait()
            cd.wait()
            cb.wait()
            cc.wait()
            @pl.when(p > 0)
            def _():
                oa, ob_ = out_copies(ch - 2, slot)
                oa.wait()
                ob_.wait()

            compute_chunk(slot)
            oa, ob_ = out_copies(ch, slot)
            oa.start()
            ob_.start()
            @pl.when(ch + 2 < nch)
            def _():
                for cp in in_copies(ch + 2, slot):
                    cp.start()

        def pair_body(p, _):
            ch0 = p * 2
            half(p, ch0, 0)
            half(p, ch0 + 1, 1)
            return 0

        for cp in in_copies(0, 0):
            cp.start()
        for cp in in_copies(1, 1):
            cp.start()
        lax.fori_loop(0, npair, pair_body, 0)
        fa, fb = out_copies(nch - 2, 0)
        fa.wait()
        fb.wait()
        ga, gb = out_copies(nch - 1, 1)
        ga.wait()
        gb.wait()

    return body(op2, dep2, b1_f, b2_f)


def kernel(opacities, values, depths, origins, dirs):
    n = opacities.shape[0]
    b1 = (jnp.pad(values[:, :, 0], ((0, 0), (0, 64)))
          + jnp.pad(values[:, :, 1], ((0, 0), (64, 0))))
    b2 = (jnp.pad(values[:, :, 2], ((0, 0), (0, 64)))
          + jnp.pad(origins, ((0, 0), (64, 61)))
          + jnp.pad(dirs, ((0, 0), (67, 58))))
    y1, y2 = _sc_call(n, opacities, depths, b1.reshape(-1), b2.reshape(-1))
    y1 = y1.reshape(n, 128)
    y2 = y2.reshape(n, 128)
    accv = y2[:, 64:67][:, None, :]
    acco = jnp.broadcast_to(y2[:, 67:68][:, None, :], (n, 1, 3))
    coords = jnp.stack([y1[:, 0:64], y1[:, 64:128], y2[:, 0:64]], axis=-1)
    return jnp.concatenate([accv, acco, coords], axis=1)


# speedup vs baseline: 1.0190x; 1.0117x over previous
import functools

import jax
import jax.numpy as jnp
from jax import lax
from jax.experimental import pallas as pl
from jax.experimental.pallas import tpu as pltpu
from jax.experimental.pallas import tpu_sc as plsc

_PTS = 64
_IMP = 64
_EPS = 1e-5
_CH = 64
_L = 16


def _sc_call(n, opacities, depths, b1_f, b2_f):
    info = plsc.get_sparse_core_info()
    nc, ns = info.num_cores, info.num_subcores
    nw = nc * ns
    rpw = n // nw
    nch = rpw // _CH
    npair = nch // 2
    groups = _CH // _L
    csz = _CH * 128
    cdfg = _PTS * _L
    histg = (_IMP + 2) * _L

    mesh = plsc.VectorSubcoreMesh(core_axis_name="c", subcore_axis_name="s")

    @functools.partial(
        pl.kernel,
        out_type=[jax.ShapeDtypeStruct((n, _PTS), jnp.float32),
                  jax.ShapeDtypeStruct((n, _PTS), jnp.float32),
                  jax.ShapeDtypeStruct((n, _PTS), jnp.float32),
                  jax.ShapeDtypeStruct((n, 4), jnp.float32)],
        mesh=mesh,
        scratch_types=[
            pltpu.VMEM((2 * _CH, _PTS), jnp.float32),
            pltpu.VMEM((2 * _CH, _PTS), jnp.float32),
            pltpu.VMEM((2 * csz,), jnp.float32),
            pltpu.VMEM((2 * csz,), jnp.float32),
            pltpu.VMEM((_CH, _PTS), jnp.float32),
            pltpu.VMEM((_CH, _PTS), jnp.float32),
            pltpu.VMEM((_CH, _PTS), jnp.float32),
            pltpu.VMEM((_CH, 4), jnp.float32),
            pltpu.VMEM((4 * cdfg,), jnp.float32),
            pltpu.VMEM((4 * histg,), jnp.int32),
            pltpu.SemaphoreType.DMA,
            pltpu.SemaphoreType.DMA,
            pltpu.SemaphoreType.DMA,
        ],
        compiler_params=pltpu.CompilerParams(needs_layout_passes=False),
    )
    def body(op_h, dep_h, b1_h, b2_h, yx_h, yy_h, yz_h, aq_h,
             op_vm, dep_vm, b1_vm, b2_vm, yx_vm, yy_vm, yz_vm, aq_vm,
             cdf_vm, hist_vm, si0, si1, so):
        wid = lax.axis_index("s") * nc + lax.axis_index("c")
        iota = lax.iota(jnp.int32, _L)
        zf = jnp.zeros((_L,), jnp.float32)
        onef = jnp.ones((_L,), jnp.float32)
        onei = jnp.ones((_L,), jnp.int32)
        zi = jnp.zeros((_L,), jnp.int32)
        si = (si0, si1)

        def in_copies(ch, slot):
            rows = pl.ds(wid * rpw + ch * _CH, _CH)
            off = pl.ds(wid * rpw * 128 + ch * csz, csz)
            v = pl.ds(slot * _CH, _CH)
            b = pl.ds(slot * csz, csz)
            s = si[slot]
            return (
                pltpu.make_async_copy(op_h.at[rows], op_vm.at[v], s),
                pltpu.make_async_copy(dep_h.at[rows], dep_vm.at[v], s),
                pltpu.make_async_copy(b1_h.at[off], b1_vm.at[b], s),
                pltpu.make_async_copy(b2_h.at[off], b2_vm.at[b], s),
            )

        def out_copies(ch):
            rows = pl.ds(wid * rpw + ch * _CH, _CH)
            return (
                pltpu.make_async_copy(yx_vm, yx_h.at[rows], so),
                pltpu.make_async_copy(yy_vm, yy_h.at[rows], so),
                pltpu.make_async_copy(yz_vm, yz_h.at[rows], so),
                pltpu.make_async_copy(aq_vm, aq_h.at[rows], so),
            )

        def compute_group(g, slot):
            row = slot * _CH + g * _L + iota
            grow = g * _L + iota
            ray_b = slot * csz + grow * 128
            coff = g * cdfg
            hoff = g * histg

            @plsc.parallel_loop(0, _IMP + 2, unroll=8)
            def hzero(v):
                hist_vm[pl.ds(hoff + v * _L, _L)] = zi

            ob = ray_b + 64
            o0 = plsc.load_gather(b2_vm, [ob])
            o1 = plsc.load_gather(b2_vm, [ob + 1])
            o2 = plsc.load_gather(b2_vm, [ob + 2])
            e0 = plsc.load_gather(b2_vm, [ob + 3])
            e1 = plsc.load_gather(b2_vm, [ob + 4])
            e2 = plsc.load_gather(b2_vm, [ob + 5])

            @plsc.parallel_loop(0, _PTS, unroll=8,
                               carry=(onef, zf, zf, zf, zf))
            def march(k, carry):
                trans, cdf, a0, a1, a2 = carry
                opk = plsc.load_gather(op_vm, [row, zi + k])
                w = opk * trans
                trans = trans * (1.0 - opk)
                cdf = cdf + (w + _EPS)
                cdf_vm[pl.ds(coff + k * _L, _L)] = cdf
                vb = ray_b + k
                a0 = a0 + w * plsc.load_gather(b1_vm, [vb])
                a1 = a1 + w * plsc.load_gather(b1_vm, [vb + 64])
                a2 = a2 + w * plsc.load_gather(b2_vm, [vb])
                return trans, cdf, a0, a1, a2

            _, ctot, a0, a1, a2 = march
            acc_o = jnp.clip(ctot - _PTS * _EPS, 0.0, 1.0)
            plsc.store_scatter(aq_vm, [grow, zi], a0)
            plsc.store_scatter(aq_vm, [grow, zi + 1], a1)
            plsc.store_scatter(aq_vm, [grow, zi + 2], a2)
            plsc.store_scatter(aq_vm, [grow, zi + 3], acc_o)

            scale = jnp.float32(_IMP) / ctot

            @plsc.parallel_loop(0, _PTS, unroll=8)
            def bink(k):
                ck = cdf_vm[pl.ds(coff + k * _L, _L)]
                x = ck * scale
                xi = x.astype(jnp.int32)
                xi = xi + (xi.astype(jnp.float32) < x).astype(jnp.int32)
                m = jnp.minimum(xi, _IMP + 1)
                plsc.addupdate_scatter(hist_vm, [hoff + m * _L + iota], onei)

            c_first = cdf_vm[pl.ds(coff, _L)]
            d_first = plsc.load_gather(dep_vm, [row, zi])
            d_last = plsc.load_gather(dep_vm, [row, zi + (_PTS - 1)])

            @plsc.parallel_loop(1, _IMP + 1, unroll=8,
                               carry=(d_first, hist_vm[pl.ds(hoff, _L)]))
            def sample(j, carry):
                f_prev, cnt = carry
                cnt = cnt + hist_vm[pl.ds(hoff + j * _L, _L)]
                i = jnp.clip(cnt, 1, _PTS - 1)
                g0 = coff + (i - 1) * _L + iota
                c0 = plsc.load_gather(cdf_vm, [g0])
                c1 = plsc.load_gather(cdf_vm, [g0 + _L])
                d0 = plsc.load_gather(dep_vm, [row, i - 1])
                d1 = plsc.load_gather(dep_vm, [row, i])
                uj = lax.convert_element_type(j, jnp.float32) * (1.0 / _IMP)
                u = uj * ctot
                f = d0 + ((u - c0) / (c1 - c0)) * (d1 - d0)
                f = jnp.where(u < c_first, d_first, f)
                f = jnp.where(u >= ctot, d_last, f)
                mid = 0.5 * (f_prev + f)
                jm = zi + (j - 1)
                plsc.store_scatter(yx_vm, [grow, jm], o0 + mid * e0)
                plsc.store_scatter(yy_vm, [grow, jm], o1 + mid * e1)
                plsc.store_scatter(yz_vm, [grow, jm], o2 + mid * e2)
                return f, cnt

            del sample

        def compute_chunk(slot):
            @plsc.parallel_loop(0, groups, unroll=2)
            def grp(g):
                compute_group(g, slot)

        def half(ch, slot, has_prev_out):
            for cp in in_copies(ch, slot):
                cp.wait()
            @pl.when(has_prev_out)
            def _():
                for cp in out_copies(ch - 1):
                    cp.wait()

            compute_chunk(slot)
            for cp in out_copies(ch):
                cp.start()
            @pl.when(ch + 2 < nch)
            def _():
                for cp in in_copies(ch + 2, slot):
                    cp.start()

        def pair_body(p, _):
            ch0 = p * 2
            half(ch0, 0, p > 0)
            half(ch0 + 1, 1, ch0 + 1 > 0)
            return 0

        for cp in in_copies(0, 0):
            cp.start()
        for cp in in_copies(1, 1):
            cp.start()
        lax.fori_loop(0, npair, pair_body, 0)
        for cp in out_copies(nch - 1):
            cp.wait()

    return body(opacities, depths, b1_f, b2_f)


def kernel(opacities, values, depths, origins, dirs):
    n = opacities.shape[0]
    b1 = (jnp.pad(values[:, :, 0], ((0, 0), (0, 64)))
          + jnp.pad(values[:, :, 1], ((0, 0), (64, 0))))
    b2 = (jnp.pad(values[:, :, 2], ((0, 0), (0, 64)))
          + jnp.pad(origins, ((0, 0), (64, 61)))
          + jnp.pad(dirs, ((0, 0), (67, 58))))
    yx, yy, yz, accq = _sc_call(n, opacities, depths,
                                b1.reshape(-1), b2.reshape(-1))
    coords = jnp.stack([yx, yy, yz], axis=-1)
    return jnp.concatenate(
        [accq[:, None, 0:3],
         jnp.broadcast_to(accq[:, 3:4, None], (n, 1, 3)),
         coords], axis=1)
